# Initial kernel scaffold; baseline (speedup 1.0000x reference)
import jax
import jax.numpy as jnp
from jax.experimental import pallas as pl


def kernel(doy, pe):
    raise NotImplementedError("write your pallas kernel here")


# speedup vs baseline: 3.6227x; 3.6227x over previous
import functools

import jax
import jax.numpy as jnp
from jax import lax
from jax.experimental import pallas as pl
from jax.experimental.pallas import tpu as pltpu
from jax.experimental.pallas import tpu_sc as plsc

D_MODEL = 512
NUM_CORES = 2
NUM_SUBCORES = 16
NUM_WORKERS = NUM_CORES * NUM_SUBCORES
CHUNK = 80


@functools.partial(jax.jit, static_argnums=(2,))
def _gather_rows(pe, doy3, nch):
    n_rows = NUM_WORKERS * nch * CHUNK
    per_w = nch * CHUNK
    mesh = plsc.VectorSubcoreMesh(
        core_axis_name="c", subcore_axis_name="s",
        num_cores=NUM_CORES, num_subcores=NUM_SUBCORES)

    @functools.partial(
        pl.kernel,
        mesh=mesh,
        out_type=jax.ShapeDtypeStruct((n_rows, D_MODEL), jnp.float32),
        scratch_types=[
            pltpu.VMEM((nch, CHUNK), jnp.int32),
            pltpu.VMEM((CHUNK, D_MODEL), jnp.float32),
            pltpu.VMEM((CHUNK, D_MODEL), jnp.float32),
            pltpu.SemaphoreType.DMA,
            pltpu.SemaphoreType.DMA,
            pltpu.SemaphoreType.DMA,
            pltpu.SemaphoreType.DMA,
        ],
    )
    def k(pe_hbm, doy_hbm, out_hbm, idx_v, buf0, buf1, g0, g1, o0, o1):
        wid = lax.axis_index("s") * NUM_CORES + lax.axis_index("c")
        base = wid * per_w
        bufs = (buf0, buf1)
        gsems = (g0, g1)
        osems = (o0, o1)

        pltpu.sync_copy(doy_hbm.at[wid], idx_v)
        for b in range(2):
            pltpu.make_async_copy(
                pe_hbm.at[idx_v.at[b]], bufs[b], gsems[b]).start()

        def body(gp, carry):
            for b in range(2):
                g = gp * 2 + b
                pltpu.make_async_copy(
                    pe_hbm.at[idx_v.at[g]], bufs[b], gsems[b]).wait()
                out_slice = out_hbm.at[pl.ds(base + g * CHUNK, CHUNK)]
                pltpu.make_async_copy(bufs[b], out_slice, osems[b]).start()

                @pl.when(g + 2 < nch)
                def _():
                    pltpu.make_async_copy(bufs[b], out_slice, osems[b]).wait()
                    pltpu.make_async_copy(
                        pe_hbm.at[idx_v.at[g + 2]], bufs[b], gsems[b]).start()

            return carry

        lax.fori_loop(0, nch // 2, body, 0)
        for b in range(2):
            pltpu.make_async_copy(
                bufs[b], out_hbm.at[pl.ds(base, CHUNK)], osems[b]).wait()

    return k(pe, doy3)


def kernel(doy, pe):
    batch, hist = doy.shape
    n_rows = batch * hist
    per_w = n_rows // NUM_WORKERS
    nch = per_w // CHUNK
    doy3 = doy.reshape(NUM_WORKERS, nch, CHUNK).astype(jnp.int32)
    out = _gather_rows(pe, doy3, nch)
    return out.reshape(batch, hist, pe.shape[1])
